# baseline (speedup 1.0000x reference)
import functools

import jax
import jax.numpy as jnp
from jax.experimental import pallas as pl
from jax.experimental.pallas import tpu as pltpu
from jax.experimental.pallas import tpu_sc as plsc

T = 1024
D = 768
E = 64
F = 128
BT = 64
P = 5120
G = P // BT
Z_COEF = 1e-3


def _plan_body(x_ref, gw_ref, bias_ref,
               z_ref, g16_ref, slot_ref, be_ref):
    x = x_ref[...]
    gw = gw_ref[...]
    logits = jax.lax.dot_general(x, gw, (((1,), (1,)), ((), ())),
                                 preferred_element_type=jnp.float32)
    m = jnp.max(logits, axis=1, keepdims=True)
    lse = m + jnp.log(jnp.sum(jnp.exp(logits - m), axis=1, keepdims=True))
    z_ref[...] = (Z_COEF * jnp.mean(lse * lse)).reshape(1, 1)

    affinity = jax.nn.sigmoid(logits)
    scores = affinity + bias_ref[...]
    smax = jnp.max(scores, axis=1, keepdims=True)
    lane = jax.lax.broadcasted_iota(jnp.int32, (T, E), 1)
    sel = jnp.min(jnp.where(scores >= smax, lane, E), axis=1, keepdims=True)
    onehot_b = lane == sel
    onehot = onehot_b.astype(jnp.float32)
    aff_sel = jnp.sum(jnp.where(onehot_b, affinity, 0.0), axis=1, keepdims=True)
    gating = aff_sel / (aff_sel + 1e-9)
    g16_ref[...] = jnp.broadcast_to(gating, (T, 128))

    counts = jnp.sum(onehot, axis=0, keepdims=True)
    r_i = jax.lax.broadcasted_iota(jnp.int32, (T, T), 0)
    c_i = jax.lax.broadcasted_iota(jnp.int32, (T, T), 1)
    tril = (c_i <= r_i).astype(jnp.float32)
    incl = jax.lax.dot_general(tril, onehot, (((1,), (0,)), ((), ())),
                               preferred_element_type=jnp.float32)
    rank = jnp.sum(jnp.where(onehot_b, incl, 0.0), axis=1, keepdims=True) - 1.0
    nb = jnp.floor((counts + (BT - 1)) * (1.0 / BT))
    e_r = jax.lax.broadcasted_iota(jnp.int32, (E, E), 0)
    e_c = jax.lax.broadcasted_iota(jnp.int32, (E, E), 1)
    tril_excl = (e_r < e_c).astype(jnp.float32)
    bstart = jax.lax.dot_general(nb, tril_excl, (((1,), (0,)), ((), ())),
                                 preferred_element_type=jnp.float32)
    bsel = jnp.sum(jnp.where(onehot_b, jnp.broadcast_to(bstart, (T, E)), 0.0),
                   axis=1, keepdims=True)
    slot_ref[...] = (BT * bsel + rank).astype(jnp.int32)

    used = jnp.sum(nb).astype(jnp.int32)
    g_i = jax.lax.broadcasted_iota(jnp.int32, (G, E), 0)
    d = g_i - used
    j_i = jnp.where(d < 0, g_i, g_i - KS * ((d >> int(KS).bit_length() - 1) + 1))
    bstart_i = bstart.astype(jnp.int32)
    be_ref[...] = (jnp.sum((bstart_i <= j_i).astype(jnp.int32), axis=1,
                           keepdims=True) - 1)


_plan_call = pl.pallas_call(
    _plan_body,
    out_shape=(
        jax.ShapeDtypeStruct((1, 1), jnp.float32),
        jax.ShapeDtypeStruct((T, 128), jnp.float32),
        jax.ShapeDtypeStruct((T, 1), jnp.int32),
        jax.ShapeDtypeStruct((G, 1), jnp.int32),
    ),
)


def _shared_body(x_ref, sgu_ref, sd_ref, shared_ref):
    x = x_ref[...]
    sc = jax.lax.dot_general(x, sgu_ref[...], (((1,), (1,)), ((), ())),
                             preferred_element_type=jnp.float32)
    sg = sc[:, :F]
    su = sc[:, F:]
    h = sg * jax.nn.sigmoid(sg) * su
    shared_ref[...] = jax.lax.dot_general(h, sd_ref[...], (((1,), (1,)), ((), ())),
                                          preferred_element_type=jnp.float32)


_shared_call = pl.pallas_call(
    _shared_body,
    out_shape=jax.ShapeDtypeStruct((T, D), jnp.float32),
)


def _sub_ffn(xb, gua, gub, dna, dnb):
    gate = jax.lax.dot_general(xb, gua[0], (((1,), (1,)), ((), ())),
                               preferred_element_type=jnp.float32)
    up = jax.lax.dot_general(xb, gub[0], (((1,), (1,)), ((), ())),
                             preferred_element_type=jnp.float32)
    h = gate * jax.nn.sigmoid(gate) * up
    oa = jax.lax.dot_general(h, dna[0], (((1,), (1,)), ((), ())),
                             preferred_element_type=jnp.float32)
    ob = jax.lax.dot_general(h, dnb[0], (((1,), (1,)), ((), ())),
                             preferred_element_type=jnp.float32)
    return oa, ob


KS = 8


def _ffn_body(be_ref, xs_ref, *refs):
    wrefs = refs[:-1]
    out_ref = refs[-1]
    xb = xs_ref[...]
    for k in range(KS):
        gua, gub, dna, dnb = wrefs[4 * k : 4 * k + 4]
        oa, ob = _sub_ffn(xb[k * BT : (k + 1) * BT], gua, gub, dna, dnb)
        out_ref[k * BT : (k + 1) * BT, : D // 2] = oa
        out_ref[k * BT : (k + 1) * BT, D // 2 :] = ob


def _wspecs(k):
    return [
        pl.BlockSpec((1, F, D), lambda g, be, k=k: (be[KS * g + k], 0, 0)),
        pl.BlockSpec((1, F, D), lambda g, be, k=k: (be[KS * g + k], 1, 0)),
        pl.BlockSpec((1, D // 2, F), lambda g, be, k=k: (be[KS * g + k], 0, 0)),
        pl.BlockSpec((1, D // 2, F), lambda g, be, k=k: (be[KS * g + k], 1, 0)),
    ]


_ffn_call = pl.pallas_call(
    _ffn_body,
    grid_spec=pltpu.PrefetchScalarGridSpec(
        num_scalar_prefetch=1,
        grid=(G // KS,),
        in_specs=[pl.BlockSpec((KS * BT, D), lambda g, be: (g, 0))]
        + [spec for k in range(KS) for spec in _wspecs(k)],
        out_specs=pl.BlockSpec((KS * BT, D), lambda g, be: (g, 0)),
    ),
    out_shape=jax.ShapeDtypeStruct((P, D), jnp.float32),
)


_NW = 32
_CHUNK = T // _NW


@functools.lru_cache(maxsize=None)
def _sc_calls():
    mesh1 = plsc.VectorSubcoreMesh(core_axis_name="c", subcore_axis_name="s")
    mesh2 = plsc.VectorSubcoreMesh(core_axis_name="c", subcore_axis_name="s")

    @functools.partial(
        pl.kernel,
        mesh=mesh1,
        out_type=jax.ShapeDtypeStruct((P, D), jnp.float32),
        scratch_types=[
            pltpu.VMEM((_CHUNK,), jnp.int32),
            pltpu.VMEM((_CHUNK, D), jnp.float32),
            pltpu.SemaphoreType.DMA,
            pltpu.SemaphoreType.DMA,
        ],
    )
    def _scatter_call(x_hbm, slot_hbm, xs_hbm, slot_v, x_v, sem1, sem2):
        wid = jax.lax.axis_index("s") * 2 + jax.lax.axis_index("c")
        base = wid * _CHUNK
        c1 = pltpu.async_copy(slot_hbm.at[pl.ds(base, _CHUNK)], slot_v, sem1)
        c2 = pltpu.async_copy(x_hbm.at[pl.ds(base, _CHUNK)], x_v, sem2)
        c1.wait()
        c2.wait()
        pltpu.async_copy(x_v, xs_hbm.at[slot_v], sem1).wait()

    @functools.partial(
        pl.kernel,
        mesh=mesh2,
        out_type=jax.ShapeDtypeStruct((T, D), jnp.float32),
        scratch_types=[
            pltpu.VMEM((_CHUNK,), jnp.int32),
            pltpu.VMEM((_CHUNK, D), jnp.float32),
            pltpu.VMEM((_CHUNK, D), jnp.float32),
            pltpu.VMEM((_CHUNK, 128), jnp.float32),
            pltpu.SemaphoreType.DMA,
        ],
    )
    def _combine_call(op_hbm, slot_hbm, shared_hbm, g16_hbm, out_hbm,
                      slot_v, rows_v, sh_v, g_v, sem):
        wid = jax.lax.axis_index("s") * 2 + jax.lax.axis_index("c")
        base = wid * _CHUNK
        pltpu.sync_copy(slot_hbm.at[pl.ds(base, _CHUNK)], slot_v)
        pltpu.sync_copy(shared_hbm.at[pl.ds(base, _CHUNK)], sh_v)
        pltpu.sync_copy(g16_hbm.at[pl.ds(base, _CHUNK)], g_v)
        pltpu.async_copy(op_hbm.at[slot_v], rows_v, sem).wait()

        def body(i, carry):
            gi = g_v[i, pl.ds(0, 16)]
            for j in range(D // 16):
                sl = pl.ds(j * 16, 16)
                rows_v[i, sl] = rows_v[i, sl] * gi + sh_v[i, sl]
            return carry

        jax.lax.fori_loop(0, _CHUNK, body, 0)
        pltpu.sync_copy(rows_v, out_hbm.at[pl.ds(base, _CHUNK)])

    return _scatter_call, _combine_call


def kernel(x, gate_w, expert_bias, gate_up_weight, down_weight,
           shared_gate_up, shared_down):
    z, g16, slot_col, be_col = _plan_call(x, gate_w, expert_bias.reshape(1, E))
    shared = _shared_call(x, shared_gate_up, shared_down)
    slot = slot_col.reshape(T)
    be = be_col.reshape(G)

    scatter_call, combine_call = _sc_calls()
    xs = scatter_call(x, slot)
    w4 = (gate_up_weight, gate_up_weight, down_weight, down_weight)
    out_padded = _ffn_call(be, xs, *(w4 * KS))
    out = combine_call(out_padded, slot, shared, g16)
    return out, z.reshape(())


# speedup vs baseline: 1.0222x; 1.0222x over previous
import functools

import jax
import jax.numpy as jnp
from jax.experimental import pallas as pl
from jax.experimental.pallas import tpu as pltpu
from jax.experimental.pallas import tpu_sc as plsc

T = 1024
D = 768
E = 64
F = 128
BT = 64
P = 5120
G = P // BT
Z_COEF = 1e-3


def _plan_body(x_ref, gw_ref, bias_ref, sgu_ref, sd_ref,
               z_ref, g16_ref, slot_ref, be_ref, shared_ref):
    x = x_ref[...]
    gw = gw_ref[...]
    logits = jax.lax.dot_general(x, gw, (((1,), (1,)), ((), ())),
                                 preferred_element_type=jnp.float32)
    m = jnp.max(logits, axis=1, keepdims=True)
    lse = m + jnp.log(jnp.sum(jnp.exp(logits - m), axis=1, keepdims=True))
    z_ref[...] = (Z_COEF * jnp.mean(lse * lse)).reshape(1, 1)

    affinity = jax.nn.sigmoid(logits)
    scores = affinity + bias_ref[...]
    smax = jnp.max(scores, axis=1, keepdims=True)
    lane = jax.lax.broadcasted_iota(jnp.int32, (T, E), 1)
    sel = jnp.min(jnp.where(scores >= smax, lane, E), axis=1, keepdims=True)
    onehot_b = lane == sel
    onehot = onehot_b.astype(jnp.float32)
    aff_sel = jnp.sum(jnp.where(onehot_b, affinity, 0.0), axis=1, keepdims=True)
    gating = aff_sel / (aff_sel + 1e-9)
    g16_ref[...] = jnp.broadcast_to(gating, (T, 128))

    counts = jnp.sum(onehot, axis=0, keepdims=True)
    r_i = jax.lax.broadcasted_iota(jnp.int32, (T, T), 0)
    c_i = jax.lax.broadcasted_iota(jnp.int32, (T, T), 1)
    tril = (c_i <= r_i).astype(jnp.float32)
    incl = jax.lax.dot_general(tril, onehot, (((1,), (0,)), ((), ())),
                               preferred_element_type=jnp.float32)
    rank = jnp.sum(jnp.where(onehot_b, incl, 0.0), axis=1, keepdims=True) - 1.0
    nb = jnp.floor((counts + (BT - 1)) * (1.0 / BT))
    e_r = jax.lax.broadcasted_iota(jnp.int32, (E, E), 0)
    e_c = jax.lax.broadcasted_iota(jnp.int32, (E, E), 1)
    tril_excl = (e_r < e_c).astype(jnp.float32)
    bstart = jax.lax.dot_general(nb, tril_excl, (((1,), (0,)), ((), ())),
                                 preferred_element_type=jnp.float32)
    bsel = jnp.sum(jnp.where(onehot_b, jnp.broadcast_to(bstart, (T, E)), 0.0),
                   axis=1, keepdims=True)
    slot_ref[...] = (BT * bsel + rank).astype(jnp.int32)

    used = jnp.sum(nb).astype(jnp.int32)
    g_i = jax.lax.broadcasted_iota(jnp.int32, (G, E), 0)
    d = g_i - used
    j_i = jnp.where(d < 0, g_i, g_i - KS * ((d >> int(KS).bit_length() - 1) + 1))
    bstart_i = bstart.astype(jnp.int32)
    be_ref[...] = (jnp.sum((bstart_i <= j_i).astype(jnp.int32), axis=1,
                           keepdims=True) - 1)

    sc = jax.lax.dot_general(x, sgu_ref[...], (((1,), (1,)), ((), ())),
                             preferred_element_type=jnp.float32)
    sg = sc[:, :F]
    su = sc[:, F:]
    h = sg * jax.nn.sigmoid(sg) * su
    shared_ref[...] = jax.lax.dot_general(h, sd_ref[...], (((1,), (1,)), ((), ())),
                                          preferred_element_type=jnp.float32)


_plan_call = pl.pallas_call(
    _plan_body,
    out_shape=(
        jax.ShapeDtypeStruct((1, 1), jnp.float32),
        jax.ShapeDtypeStruct((T, 128), jnp.float32),
        jax.ShapeDtypeStruct((T, 1), jnp.int32),
        jax.ShapeDtypeStruct((G, 1), jnp.int32),
        jax.ShapeDtypeStruct((T, D), jnp.float32),
    ),
)


def _sub_ffn(xb, gua, gub, dna, dnb):
    gate = jax.lax.dot_general(xb, gua[0], (((1,), (1,)), ((), ())),
                               preferred_element_type=jnp.float32)
    up = jax.lax.dot_general(xb, gub[0], (((1,), (1,)), ((), ())),
                             preferred_element_type=jnp.float32)
    h = gate * jax.nn.sigmoid(gate) * up
    oa = jax.lax.dot_general(h, dna[0], (((1,), (1,)), ((), ())),
                             preferred_element_type=jnp.float32)
    ob = jax.lax.dot_general(h, dnb[0], (((1,), (1,)), ((), ())),
                             preferred_element_type=jnp.float32)
    return oa, ob


KS = 16


def _ffn_body(be_ref, xs_ref, *refs):
    wrefs = refs[:-1]
    out_ref = refs[-1]
    xb = xs_ref[...]
    for k in range(KS):
        gua, gub, dna, dnb = wrefs[4 * k : 4 * k + 4]
        oa, ob = _sub_ffn(xb[k * BT : (k + 1) * BT], gua, gub, dna, dnb)
        out_ref[k * BT : (k + 1) * BT, : D // 2] = oa
        out_ref[k * BT : (k + 1) * BT, D // 2 :] = ob


def _wspecs(k):
    return [
        pl.BlockSpec((1, F, D), lambda g, be, k=k: (be[KS * g + k], 0, 0)),
        pl.BlockSpec((1, F, D), lambda g, be, k=k: (be[KS * g + k], 1, 0)),
        pl.BlockSpec((1, D // 2, F), lambda g, be, k=k: (be[KS * g + k], 0, 0)),
        pl.BlockSpec((1, D // 2, F), lambda g, be, k=k: (be[KS * g + k], 1, 0)),
    ]


_ffn_call = pl.pallas_call(
    _ffn_body,
    grid_spec=pltpu.PrefetchScalarGridSpec(
        num_scalar_prefetch=1,
        grid=(G // KS,),
        in_specs=[pl.BlockSpec((KS * BT, D), lambda g, be: (g, 0))]
        + [spec for k in range(KS) for spec in _wspecs(k)],
        out_specs=pl.BlockSpec((KS * BT, D), lambda g, be: (g, 0)),
    ),
    out_shape=jax.ShapeDtypeStruct((P, D), jnp.float32),
)


_NW = 32
_CHUNK = T // _NW


@functools.lru_cache(maxsize=None)
def _sc_calls():
    mesh1 = plsc.VectorSubcoreMesh(core_axis_name="c", subcore_axis_name="s")
    mesh2 = plsc.VectorSubcoreMesh(core_axis_name="c", subcore_axis_name="s")

    @functools.partial(
        pl.kernel,
        mesh=mesh1,
        out_type=jax.ShapeDtypeStruct((P, D), jnp.float32),
        scratch_types=[
            pltpu.VMEM((_CHUNK,), jnp.int32),
            pltpu.VMEM((_CHUNK, D), jnp.float32),
            pltpu.SemaphoreType.DMA,
            pltpu.SemaphoreType.DMA,
        ],
    )
    def _scatter_call(x_hbm, slot_hbm, xs_hbm, slot_v, x_v, sem1, sem2):
        wid = jax.lax.axis_index("s") * 2 + jax.lax.axis_index("c")
        base = wid * _CHUNK
        c1 = pltpu.async_copy(slot_hbm.at[pl.ds(base, _CHUNK)], slot_v, sem1)
        c2 = pltpu.async_copy(x_hbm.at[pl.ds(base, _CHUNK)], x_v, sem2)
        c1.wait()
        c2.wait()
        pltpu.async_copy(x_v, xs_hbm.at[slot_v], sem1).wait()

    @functools.partial(
        pl.kernel,
        mesh=mesh2,
        out_type=jax.ShapeDtypeStruct((T, D), jnp.float32),
        scratch_types=[
            pltpu.VMEM((_CHUNK,), jnp.int32),
            pltpu.VMEM((_CHUNK, D), jnp.float32),
            pltpu.VMEM((_CHUNK, D), jnp.float32),
            pltpu.VMEM((_CHUNK, 128), jnp.float32),
            pltpu.SemaphoreType.DMA,
        ],
    )
    def _combine_call(op_hbm, slot_hbm, shared_hbm, g16_hbm, out_hbm,
                      slot_v, rows_v, sh_v, g_v, sem):
        wid = jax.lax.axis_index("s") * 2 + jax.lax.axis_index("c")
        base = wid * _CHUNK
        pltpu.sync_copy(slot_hbm.at[pl.ds(base, _CHUNK)], slot_v)
        pltpu.sync_copy(shared_hbm.at[pl.ds(base, _CHUNK)], sh_v)
        pltpu.sync_copy(g16_hbm.at[pl.ds(base, _CHUNK)], g_v)
        pltpu.async_copy(op_hbm.at[slot_v], rows_v, sem).wait()

        def body(i, carry):
            gi = g_v[i, pl.ds(0, 16)]
            for j in range(D // 16):
                sl = pl.ds(j * 16, 16)
                rows_v[i, sl] = rows_v[i, sl] * gi + sh_v[i, sl]
            return carry

        jax.lax.fori_loop(0, _CHUNK, body, 0)
        pltpu.sync_copy(rows_v, out_hbm.at[pl.ds(base, _CHUNK)])

    return _scatter_call, _combine_call


def kernel(x, gate_w, expert_bias, gate_up_weight, down_weight,
           shared_gate_up, shared_down):
    z, g16, slot_col, be_col, shared = _plan_call(
        x, gate_w, expert_bias.reshape(1, E), shared_gate_up, shared_down)
    slot = slot_col.reshape(T)
    be = be_col.reshape(G)

    scatter_call, combine_call = _sc_calls()
    xs = scatter_call(x, slot)
    w4 = (gate_up_weight, gate_up_weight, down_weight, down_weight)
    out_padded = _ffn_call(be, xs, *(w4 * KS))
    out = combine_call(out_padded, slot, shared, g16)
    return out, z.reshape(())


# speedup vs baseline: 1.0275x; 1.0051x over previous
import functools

import jax
import jax.numpy as jnp
from jax.experimental import pallas as pl
from jax.experimental.pallas import tpu as pltpu
from jax.experimental.pallas import tpu_sc as plsc

T = 1024
D = 768
E = 64
F = 128
BT = 64
P = 5120
G = P // BT
Z_COEF = 1e-3


def _plan_body(x_ref, gw_ref, bias_ref, sgu_ref, sd_ref,
               z_ref, g16_ref, slot_ref, be_ref, shared_ref):
    x = x_ref[...]
    gw = gw_ref[...]
    logits = jax.lax.dot_general(x, gw, (((1,), (1,)), ((), ())),
                                 preferred_element_type=jnp.float32)
    m = jnp.max(logits, axis=1, keepdims=True)
    lse = m + jnp.log(jnp.sum(jnp.exp(logits - m), axis=1, keepdims=True))
    z_ref[...] = (Z_COEF * jnp.mean(lse * lse)).reshape(1, 1)

    affinity = jax.nn.sigmoid(logits)
    scores = affinity + bias_ref[...]
    smax = jnp.max(scores, axis=1, keepdims=True)
    lane = jax.lax.broadcasted_iota(jnp.int32, (T, E), 1)
    sel = jnp.min(jnp.where(scores >= smax, lane, E), axis=1, keepdims=True)
    onehot_b = lane == sel
    onehot = onehot_b.astype(jnp.float32)
    aff_sel = jnp.sum(jnp.where(onehot_b, affinity, 0.0), axis=1, keepdims=True)
    gating = aff_sel / (aff_sel + 1e-9)
    g16_ref[...] = jnp.broadcast_to(gating, (T, 128))

    counts = jnp.sum(onehot, axis=0, keepdims=True)
    r_i = jax.lax.broadcasted_iota(jnp.int32, (T, T), 0)
    c_i = jax.lax.broadcasted_iota(jnp.int32, (T, T), 1)
    tril = (c_i <= r_i).astype(jnp.float32)
    incl = jax.lax.dot_general(tril, onehot, (((1,), (0,)), ((), ())),
                               preferred_element_type=jnp.float32)
    rank = jnp.sum(jnp.where(onehot_b, incl, 0.0), axis=1, keepdims=True) - 1.0
    nb = jnp.floor((counts + (BT - 1)) * (1.0 / BT))
    e_r = jax.lax.broadcasted_iota(jnp.int32, (E, E), 0)
    e_c = jax.lax.broadcasted_iota(jnp.int32, (E, E), 1)
    tril_excl = (e_r < e_c).astype(jnp.float32)
    bstart = jax.lax.dot_general(nb, tril_excl, (((1,), (0,)), ((), ())),
                                 preferred_element_type=jnp.float32)
    bsel = jnp.sum(jnp.where(onehot_b, jnp.broadcast_to(bstart, (T, E)), 0.0),
                   axis=1, keepdims=True)
    slot_ref[...] = (BT * bsel + rank).astype(jnp.int32)

    used = jnp.sum(nb).astype(jnp.int32)
    g_i = jax.lax.broadcasted_iota(jnp.int32, (G, E), 0)
    d = g_i - used
    j_i = jnp.where(d < 0, g_i, g_i - KS * ((d >> int(KS).bit_length() - 1) + 1))
    bstart_i = bstart.astype(jnp.int32)
    be_ref[...] = (jnp.sum((bstart_i <= j_i).astype(jnp.int32), axis=1,
                           keepdims=True) - 1)

    sc = jax.lax.dot_general(x, sgu_ref[...], (((1,), (1,)), ((), ())),
                             preferred_element_type=jnp.float32)
    sg = sc[:, :F]
    su = sc[:, F:]
    h = sg * jax.nn.sigmoid(sg) * su
    shared_ref[...] = jax.lax.dot_general(h, sd_ref[...], (((1,), (1,)), ((), ())),
                                          preferred_element_type=jnp.float32)


_plan_call = pl.pallas_call(
    _plan_body,
    out_shape=(
        jax.ShapeDtypeStruct((1, 1), jnp.float32),
        jax.ShapeDtypeStruct((T, 128), jnp.float32),
        jax.ShapeDtypeStruct((T, 1), jnp.int32),
        jax.ShapeDtypeStruct((G, 1), jnp.int32),
        jax.ShapeDtypeStruct((T, D), jnp.float32),
    ),
)


def _sub_ffn(xb, gua, gub, dna, dnb):
    gate = jax.lax.dot_general(xb, gua[0], (((1,), (1,)), ((), ())),
                               preferred_element_type=jnp.float32)
    up = jax.lax.dot_general(xb, gub[0], (((1,), (1,)), ((), ())),
                             preferred_element_type=jnp.float32)
    h = gate * jax.nn.sigmoid(gate) * up
    oa = jax.lax.dot_general(h, dna[0], (((1,), (1,)), ((), ())),
                             preferred_element_type=jnp.float32)
    ob = jax.lax.dot_general(h, dnb[0], (((1,), (1,)), ((), ())),
                             preferred_element_type=jnp.float32)
    return oa, ob


KS = 8


def _ffn_body(be_ref, xs_ref, *refs):
    wrefs = refs[:-1]
    out_ref = refs[-1]
    xb = xs_ref[...]
    for k in range(KS):
        gua, gub, dna, dnb = wrefs[4 * k : 4 * k + 4]
        oa, ob = _sub_ffn(xb[k * BT : (k + 1) * BT], gua, gub, dna, dnb)
        out_ref[k * BT : (k + 1) * BT, : D // 2] = oa
        out_ref[k * BT : (k + 1) * BT, D // 2 :] = ob


def _wspecs(k):
    return [
        pl.BlockSpec((1, F, D), lambda g, be, k=k: (be[KS * g + k], 0, 0)),
        pl.BlockSpec((1, F, D), lambda g, be, k=k: (be[KS * g + k], 1, 0)),
        pl.BlockSpec((1, D // 2, F), lambda g, be, k=k: (be[KS * g + k], 0, 0)),
        pl.BlockSpec((1, D // 2, F), lambda g, be, k=k: (be[KS * g + k], 1, 0)),
    ]


_ffn_call = pl.pallas_call(
    _ffn_body,
    grid_spec=pltpu.PrefetchScalarGridSpec(
        num_scalar_prefetch=1,
        grid=(G // KS,),
        in_specs=[pl.BlockSpec((KS * BT, D), lambda g, be: (g, 0))]
        + [spec for k in range(KS) for spec in _wspecs(k)],
        out_specs=pl.BlockSpec((KS * BT, D), lambda g, be: (g, 0)),
    ),
    out_shape=jax.ShapeDtypeStruct((P, D), jnp.float32),
)


_NW = 32
_CHUNK = T // _NW


@functools.lru_cache(maxsize=None)
def _sc_calls():
    mesh1 = plsc.VectorSubcoreMesh(core_axis_name="c", subcore_axis_name="s")
    mesh2 = plsc.VectorSubcoreMesh(core_axis_name="c", subcore_axis_name="s")

    @functools.partial(
        pl.kernel,
        mesh=mesh1,
        out_type=jax.ShapeDtypeStruct((P, D), jnp.float32),
        scratch_types=[
            pltpu.VMEM((_CHUNK,), jnp.int32),
            pltpu.VMEM((_CHUNK, D), jnp.float32),
            pltpu.SemaphoreType.DMA,
            pltpu.SemaphoreType.DMA,
        ],
    )
    def _scatter_call(x_hbm, slot_hbm, xs_hbm, slot_v, x_v, sem1, sem2):
        wid = jax.lax.axis_index("s") * 2 + jax.lax.axis_index("c")
        base = wid * _CHUNK
        c1 = pltpu.async_copy(slot_hbm.at[pl.ds(base, _CHUNK)], slot_v, sem1)
        c2 = pltpu.async_copy(x_hbm.at[pl.ds(base, _CHUNK)], x_v, sem2)
        c1.wait()
        c2.wait()
        pltpu.async_copy(x_v, xs_hbm.at[slot_v], sem1).wait()

    @functools.partial(
        pl.kernel,
        mesh=mesh2,
        out_type=jax.ShapeDtypeStruct((T, D), jnp.float32),
        scratch_types=[
            pltpu.VMEM((_CHUNK,), jnp.int32),
            pltpu.VMEM((_CHUNK, D), jnp.float32),
            pltpu.VMEM((_CHUNK, D), jnp.float32),
            pltpu.VMEM((_CHUNK, 128), jnp.float32),
            pltpu.SemaphoreType.DMA,
        ],
    )
    def _combine_call(op_hbm, slot_hbm, shared_hbm, g16_hbm, out_hbm,
                      slot_v, rows_v, sh_v, g_v, sem):
        wid = jax.lax.axis_index("s") * 2 + jax.lax.axis_index("c")
        base = wid * _CHUNK
        pltpu.sync_copy(slot_hbm.at[pl.ds(base, _CHUNK)], slot_v)
        pltpu.sync_copy(shared_hbm.at[pl.ds(base, _CHUNK)], sh_v)
        pltpu.sync_copy(g16_hbm.at[pl.ds(base, _CHUNK)], g_v)
        pltpu.async_copy(op_hbm.at[slot_v], rows_v, sem).wait()

        def body(i, carry):
            gi = g_v[i, pl.ds(0, 16)]
            for j in range(D // 16):
                sl = pl.ds(j * 16, 16)
                rows_v[i, sl] = rows_v[i, sl] * gi + sh_v[i, sl]
            return carry

        jax.lax.fori_loop(0, _CHUNK, body, 0)
        pltpu.sync_copy(rows_v, out_hbm.at[pl.ds(base, _CHUNK)])

    return _scatter_call, _combine_call


def kernel(x, gate_w, expert_bias, gate_up_weight, down_weight,
           shared_gate_up, shared_down):
    z, g16, slot_col, be_col, shared = _plan_call(
        x, gate_w, expert_bias.reshape(1, E), shared_gate_up, shared_down)
    slot = slot_col.reshape(T)
    be = be_col.reshape(G)

    scatter_call, combine_call = _sc_calls()
    xs = scatter_call(x, slot)
    w4 = (gate_up_weight, gate_up_weight, down_weight, down_weight)
    out_padded = _ffn_call(be, xs, *(w4 * KS))
    out = combine_call(out_padded, slot, shared, g16)
    return out, z.reshape(())
